# Initial kernel scaffold; baseline (speedup 1.0000x reference)
import jax
import jax.numpy as jnp
from jax.experimental import pallas as pl


def kernel(ids, emb, W, b):
    raise NotImplementedError("write your pallas kernel here")


# speedup vs baseline: 6.0731x; 6.0731x over previous
import jax
import jax.numpy as jnp
from jax import lax
from jax.experimental import pallas as pl
from jax.experimental.pallas import tpu as pltpu
from jax.experimental.pallas import tpu_sc as plsc

B, S, D, C = 4096, 50, 128, 1000
NC, NS = 2, 16
NW = NC * NS
BPW = B // NW
G = 2
NG = BPW // G
IDXM = G * S
NL = D // 16


def _sc_body(ids_hbm, emb_hbm, out_hbm, idx_v, rows_v, out_v, sem):
    wid = lax.axis_index("s") * NC + lax.axis_index("c")
    base = wid * BPW
    pltpu.sync_copy(ids_hbm.at[wid], idx_v)

    def group(j, carry):
        pltpu.async_copy(emb_hbm.at[idx_v.at[j]], rows_v, sem).wait()
        for g in range(G):
            def inner(r, accs):
                row = g * S + r
                return tuple(a + rows_v[row, pl.ds(c * 16, 16)]
                             for c, a in enumerate(accs))
            accs = lax.fori_loop(
                0, S, inner,
                tuple(jnp.zeros((16,), jnp.float32) for _ in range(NL)))
            for c in range(NL):
                out_v[j * G + g, pl.ds(c * 16, 16)] = accs[c]
        return carry

    lax.fori_loop(0, NG, group, 0)
    pltpu.sync_copy(out_v, out_hbm.at[pl.ds(base, BPW)])


def _sc_sum(ids_grouped, emb):
    mesh = plsc.VectorSubcoreMesh(
        core_axis_name="c", subcore_axis_name="s",
        num_cores=NC, num_subcores=NS)
    f = pl.kernel(
        _sc_body,
        out_type=jax.ShapeDtypeStruct((B, D), jnp.float32),
        mesh=mesh,
        scratch_types=[
            pltpu.VMEM((NG, IDXM), jnp.int32),
            pltpu.VMEM((IDXM, D), jnp.float32),
            pltpu.VMEM((BPW, D), jnp.float32),
            pltpu.SemaphoreType.DMA,
        ],
    )
    return f(ids_grouped, emb)


def _tc_body(sum_ref, ids_ref, w_ref, b_ref, out_ref):
    cnt = jnp.sum((ids_ref[...] != 0).astype(jnp.float32), axis=1,
                  keepdims=True)
    avg = sum_ref[...] / jnp.maximum(cnt, 1.0)
    out_ref[...] = (
        jnp.dot(avg, w_ref[...], preferred_element_type=jnp.float32)
        + b_ref[...])


def _tc_finish(summed, ids, W, b):
    bm = 512
    return pl.pallas_call(
        _tc_body,
        grid=(B // bm,),
        in_specs=[
            pl.BlockSpec((bm, D), lambda i: (i, 0)),
            pl.BlockSpec((bm, S), lambda i: (i, 0)),
            pl.BlockSpec((D, C), lambda i: (0, 0)),
            pl.BlockSpec((1, C), lambda i: (0, 0)),
        ],
        out_specs=pl.BlockSpec((bm, C), lambda i: (i, 0)),
        out_shape=jax.ShapeDtypeStruct((B, C), jnp.float32),
    )(summed, ids, W, b.reshape(1, C))


def kernel(ids, emb, W, b):
    ids = ids.astype(jnp.int32)
    ids_grouped = ids.reshape(NW, NG, IDXM)
    summed = _sc_sum(ids_grouped, emb)
    return _tc_finish(summed, ids, W, b)


# speedup vs baseline: 7.3974x; 1.2181x over previous
import jax
import jax.numpy as jnp
from jax import lax
from jax.experimental import pallas as pl
from jax.experimental.pallas import tpu as pltpu
from jax.experimental.pallas import tpu_sc as plsc

B, S, D, C = 4096, 50, 128, 1000
NC, NS = 2, 16
NW = NC * NS
BPW = B // NW
G = 2
NG = BPW // G
IDXM = G * S
NL = D // 16


def _sc_body(ids_hbm, emb_hbm, out_hbm, idx_v, rows0_v, rows1_v, out_v, sem):
    wid = lax.axis_index("s") * NC + lax.axis_index("c")
    base = wid * BPW
    pltpu.sync_copy(ids_hbm.at[wid], idx_v)
    bufs = (rows0_v, rows1_v)

    def accumulate(j, rows_v):
        for g in range(G):
            def inner(r, accs):
                row = g * S + r
                return tuple(a + rows_v[row, pl.ds(c * 16, 16)]
                             for c, a in enumerate(accs))
            accs = lax.fori_loop(
                0, S, inner,
                tuple(jnp.zeros((16,), jnp.float32) for _ in range(NL)))
            for c in range(NL):
                out_v[j * G + g, pl.ds(c * 16, 16)] = accs[c]

    def wait_gather(j, rows_v):
        pltpu.make_async_copy(emb_hbm.at[idx_v.at[j]], rows_v, sem).wait()

    pltpu.async_copy(emb_hbm.at[idx_v.at[0]], bufs[0], sem)

    def pair(p, carry):
        j = p * 2
        wait_gather(j, bufs[0])
        pltpu.async_copy(emb_hbm.at[idx_v.at[j + 1]], bufs[1], sem)
        accumulate(j, bufs[0])
        wait_gather(j + 1, bufs[1])

        @pl.when(p < NG // 2 - 1)
        def _():
            pltpu.async_copy(emb_hbm.at[idx_v.at[j + 2]], bufs[0], sem)

        accumulate(j + 1, bufs[1])
        return carry

    lax.fori_loop(0, NG // 2, pair, 0)
    pltpu.sync_copy(out_v, out_hbm.at[pl.ds(base, BPW)])


def _sc_sum(ids_grouped, emb):
    mesh = plsc.VectorSubcoreMesh(
        core_axis_name="c", subcore_axis_name="s",
        num_cores=NC, num_subcores=NS)
    f = pl.kernel(
        _sc_body,
        out_type=jax.ShapeDtypeStruct((B, D), jnp.float32),
        mesh=mesh,
        scratch_types=[
            pltpu.VMEM((NG, IDXM), jnp.int32),
            pltpu.VMEM((IDXM, D), jnp.float32),
            pltpu.VMEM((IDXM, D), jnp.float32),
            pltpu.VMEM((BPW, D), jnp.float32),
            pltpu.SemaphoreType.DMA,
        ],
    )
    return f(ids_grouped, emb)


def _tc_body(sum_ref, ids_ref, w_ref, b_ref, out_ref):
    cnt = jnp.sum((ids_ref[...] != 0).astype(jnp.float32), axis=1,
                  keepdims=True)
    avg = sum_ref[...] / jnp.maximum(cnt, 1.0)
    out_ref[...] = (
        jnp.dot(avg, w_ref[...], preferred_element_type=jnp.float32)
        + b_ref[...])


def _tc_finish(summed, ids, W, b):
    bm = 512
    return pl.pallas_call(
        _tc_body,
        grid=(B // bm,),
        in_specs=[
            pl.BlockSpec((bm, D), lambda i: (i, 0)),
            pl.BlockSpec((bm, S), lambda i: (i, 0)),
            pl.BlockSpec((D, C), lambda i: (0, 0)),
            pl.BlockSpec((1, C), lambda i: (0, 0)),
        ],
        out_specs=pl.BlockSpec((bm, C), lambda i: (i, 0)),
        out_shape=jax.ShapeDtypeStruct((B, C), jnp.float32),
    )(summed, ids, W, b.reshape(1, C))


def kernel(ids, emb, W, b):
    ids = ids.astype(jnp.int32)
    ids_grouped = ids.reshape(NW, NG, IDXM)
    summed = _sc_sum(ids_grouped, emb)
    return _tc_finish(summed, ids, W, b)


# speedup vs baseline: 7.4069x; 1.0013x over previous
import jax
import jax.numpy as jnp
from jax import lax
from jax.experimental import pallas as pl
from jax.experimental.pallas import tpu as pltpu
from jax.experimental.pallas import tpu_sc as plsc

B, S, D, C = 4096, 50, 128, 1000
NC, NS = 2, 16
NW = NC * NS
BPW = B // NW
G = 2
NG = BPW // G
IDXM = G * S
NL = D // 16


def _sc_body(ids_hbm, emb_hbm, out_hbm, idx_v, rows0_v, rows1_v, out_v, sem):
    wid = lax.axis_index("s") * NC + lax.axis_index("c")
    base = wid * BPW
    pltpu.sync_copy(ids_hbm.at[wid], idx_v)
    bufs = (rows0_v, rows1_v)

    def accumulate(j, rows_v):
        def inner(r2, accs):
            accs = list(accs)
            for dr in range(2):
                r = r2 * 2 + dr
                for g in range(G):
                    for c in range(NL):
                        accs[g * NL + c] = (accs[g * NL + c]
                                            + rows_v[g * S + r,
                                                     pl.ds(c * 16, 16)])
            return tuple(accs)

        accs = lax.fori_loop(
            0, S // 2, inner,
            tuple(jnp.zeros((16,), jnp.float32) for _ in range(G * NL)))
        for g in range(G):
            for c in range(NL):
                out_v[j * G + g, pl.ds(c * 16, 16)] = accs[g * NL + c]

    def wait_gather(j, rows_v):
        pltpu.make_async_copy(emb_hbm.at[idx_v.at[j]], rows_v, sem).wait()

    pltpu.async_copy(emb_hbm.at[idx_v.at[0]], bufs[0], sem)

    def pair(p, carry):
        j = p * 2
        wait_gather(j, bufs[0])
        pltpu.async_copy(emb_hbm.at[idx_v.at[j + 1]], bufs[1], sem)
        accumulate(j, bufs[0])
        wait_gather(j + 1, bufs[1])

        @pl.when(p < NG // 2 - 1)
        def _():
            pltpu.async_copy(emb_hbm.at[idx_v.at[j + 2]], bufs[0], sem)

        accumulate(j + 1, bufs[1])
        return carry

    lax.fori_loop(0, NG // 2, pair, 0)
    pltpu.sync_copy(out_v, out_hbm.at[pl.ds(base, BPW)])


def _sc_sum(ids_grouped, emb):
    mesh = plsc.VectorSubcoreMesh(
        core_axis_name="c", subcore_axis_name="s",
        num_cores=NC, num_subcores=NS)
    f = pl.kernel(
        _sc_body,
        out_type=jax.ShapeDtypeStruct((B, D), jnp.float32),
        mesh=mesh,
        scratch_types=[
            pltpu.VMEM((NG, IDXM), jnp.int32),
            pltpu.VMEM((IDXM, D), jnp.float32),
            pltpu.VMEM((IDXM, D), jnp.float32),
            pltpu.VMEM((BPW, D), jnp.float32),
            pltpu.SemaphoreType.DMA,
        ],
    )
    return f(ids_grouped, emb)


def _tc_body(sum_ref, ids_ref, w_ref, b_ref, out_ref):
    cnt = jnp.sum((ids_ref[...] != 0).astype(jnp.float32), axis=1,
                  keepdims=True)
    avg = sum_ref[...] / jnp.maximum(cnt, 1.0)
    out_ref[...] = (
        jnp.dot(avg, w_ref[...], preferred_element_type=jnp.float32)
        + b_ref[...])


def _tc_finish(summed, ids, W, b):
    bm = 512
    return pl.pallas_call(
        _tc_body,
        grid=(B // bm,),
        in_specs=[
            pl.BlockSpec((bm, D), lambda i: (i, 0)),
            pl.BlockSpec((bm, S), lambda i: (i, 0)),
            pl.BlockSpec((D, C), lambda i: (0, 0)),
            pl.BlockSpec((1, C), lambda i: (0, 0)),
        ],
        out_specs=pl.BlockSpec((bm, C), lambda i: (i, 0)),
        out_shape=jax.ShapeDtypeStruct((B, C), jnp.float32),
    )(summed, ids, W, b.reshape(1, C))


def kernel(ids, emb, W, b):
    ids = ids.astype(jnp.int32)
    ids_grouped = ids.reshape(NW, NG, IDXM)
    summed = _sc_sum(ids_grouped, emb)
    return _tc_finish(summed, ids, W, b)


# speedup vs baseline: 10.9791x; 1.4823x over previous
import jax
import jax.numpy as jnp
from jax import lax
from jax.experimental import pallas as pl
from jax.experimental.pallas import tpu as pltpu
from jax.experimental.pallas import tpu_sc as plsc

B, S, D, C = 4096, 50, 128, 1000
NC, NS = 2, 16
NW = NC * NS
BPW = B // NW
G = 2
NG = BPW // G
IDXM = G * S
NL = D // 16


NBUF = 4


def _sc_body(ids_hbm, emb_hbm, out_hbm, idx_v,
             rows0_v, rows1_v, rows2_v, rows3_v, out_v, sem):
    wid = lax.axis_index("s") * NC + lax.axis_index("c")
    base = wid * BPW
    pltpu.sync_copy(ids_hbm.at[wid], idx_v)
    bufs = (rows0_v, rows1_v, rows2_v, rows3_v)

    def accumulate(j, rows_v):
        def inner(r2, accs):
            accs = list(accs)
            for dr in range(2):
                r = r2 * 2 + dr
                for g in range(G):
                    for c in range(NL):
                        accs[g * NL + c] = (accs[g * NL + c]
                                            + rows_v[g * S + r,
                                                     pl.ds(c * 16, 16)])
            return tuple(accs)

        accs = lax.fori_loop(
            0, S // 2, inner,
            tuple(jnp.zeros((16,), jnp.float32) for _ in range(G * NL)))
        for g in range(G):
            for c in range(NL):
                out_v[j * G + g, pl.ds(c * 16, 16)] = accs[g * NL + c]

    def wait_gather(j, rows_v):
        pltpu.make_async_copy(emb_hbm.at[idx_v.at[j]], rows_v, sem).wait()

    for b in range(NBUF - 1):
        pltpu.async_copy(emb_hbm.at[idx_v.at[b]], bufs[b], sem)

    def ring(p, carry):
        j = p * NBUF
        for b in range(NBUF):
            wait_gather(j + b, bufs[b])
            nxt = j + b + NBUF - 1

            @pl.when(nxt < NG)
            def _():
                pltpu.async_copy(
                    emb_hbm.at[idx_v.at[nxt]], bufs[(b + NBUF - 1) % NBUF],
                    sem)

            accumulate(j + b, bufs[b])
        return carry

    lax.fori_loop(0, NG // NBUF, ring, 0)
    pltpu.sync_copy(out_v, out_hbm.at[pl.ds(base, BPW)])


def _sc_sum(ids_grouped, emb):
    mesh = plsc.VectorSubcoreMesh(
        core_axis_name="c", subcore_axis_name="s",
        num_cores=NC, num_subcores=NS)
    f = pl.kernel(
        _sc_body,
        out_type=jax.ShapeDtypeStruct((B, D), jnp.float32),
        mesh=mesh,
        scratch_types=[
            pltpu.VMEM((NG, IDXM), jnp.int32),
            pltpu.VMEM((IDXM, D), jnp.float32),
            pltpu.VMEM((IDXM, D), jnp.float32),
            pltpu.VMEM((IDXM, D), jnp.float32),
            pltpu.VMEM((IDXM, D), jnp.float32),
            pltpu.VMEM((BPW, D), jnp.float32),
            pltpu.SemaphoreType.DMA,
        ],
    )
    return f(ids_grouped, emb)


def _tc_body(sum_ref, ids_ref, w_ref, b_ref, out_ref):
    cnt = jnp.sum((ids_ref[...] != 0).astype(jnp.float32), axis=1,
                  keepdims=True)
    avg = sum_ref[...] / jnp.maximum(cnt, 1.0)
    out_ref[...] = (
        jnp.dot(avg, w_ref[...], preferred_element_type=jnp.float32)
        + b_ref[...])


def _tc_finish(summed, ids, W, b):
    bm = 512
    return pl.pallas_call(
        _tc_body,
        grid=(B // bm,),
        in_specs=[
            pl.BlockSpec((bm, D), lambda i: (i, 0)),
            pl.BlockSpec((bm, S), lambda i: (i, 0)),
            pl.BlockSpec((D, C), lambda i: (0, 0)),
            pl.BlockSpec((1, C), lambda i: (0, 0)),
        ],
        out_specs=pl.BlockSpec((bm, C), lambda i: (i, 0)),
        out_shape=jax.ShapeDtypeStruct((B, C), jnp.float32),
    )(summed, ids, W, b.reshape(1, C))


def kernel(ids, emb, W, b):
    ids = ids.astype(jnp.int32)
    ids_grouped = ids.reshape(NW, NG, IDXM)
    summed = _sc_sum(ids_grouped, emb)
    return _tc_finish(summed, ids, W, b)
